# baseline (speedup 1.0000x reference)
import functools

import jax
import jax.numpy as jnp
from jax.experimental import pallas as pl
from jax.experimental.pallas import tpu as pltpu

B, T, L = 64, 4096, 1024
H, RH, K, D = 128, 64, 1024, 64
S = 2
GRID = B // S


def _mm(a, w):
    return jax.lax.dot_general(
        a.astype(jnp.bfloat16), w.astype(jnp.bfloat16),
        (((a.ndim - 1,), (0,)), ((), ())),
        preferred_element_type=jnp.float32)


def _mm_exact(a, w):
    return jax.lax.dot_general(
        a, w, (((a.ndim - 1,), (0,)), ((), ())),
        preferred_element_type=jnp.float32,
        precision=jax.lax.Precision.HIGHEST)


def _imm(parts, wcat):
    return _mm(jnp.concatenate(parts, axis=-1), wcat)


def _sd(v):
    z = jnp.zeros_like(v[:, :1])
    return jnp.concatenate([z, v[:, :-1]], axis=1)


def _su(v):
    z = jnp.zeros_like(v[:, :1])
    return jnp.concatenate([v[:, 1:], z], axis=1)


def _resblock(x, wcat3, w2):
    r = jax.nn.relu(x)
    h = _imm([_sd(r), r, _su(r)], wcat3)
    h = jax.nn.relu(h)
    return x + _mm(h, w2)


def _rowsq(z):
    sq = z * z
    s = sq[..., 0:8]
    for g in range(1, 8):
        s = s + sq[..., 8 * g:8 * (g + 1)]
    t = s[..., 0:4] + s[..., 4:8]
    t = t[..., 0:2] + t[..., 2:4]
    return t[..., 0:1] + t[..., 1:2]


def _vqvae_kernel(x0, x1, x2, x3,
                  w1taps, b1, w2cat, b2,
                  er1c, er1w2, er2c, er2w2,
                  pvw, pvb, embT, emb, emb_sq,
                  d1c, d1b, dr1c, dr1w2, dr2c, dr2w2,
                  t1e, t1o, t1b, t2e, t2o, t2b,
                  loss_o, perp_o, idx_o, p0_o, p1_o, p2_o, p3_o,
                  counts_acc, err_acc):
    i = pl.program_id(0)

    xa0, xa1, xa2, xa3 = x0[0], x1[0], x2[0], x3[0]
    w1 = w1taps[...]
    bb1 = b1[...][None, None, :]
    ha = _mm(jnp.stack([_sd(xa3), xa0, xa1, xa2], axis=-1), w1) + bb1
    hb = _mm(jnp.stack([xa1, xa2, xa3, _su(xa0)], axis=-1), w1) + bb1
    ha = jax.nn.relu(ha)
    hb = jax.nn.relu(hb)

    h = _imm([_sd(hb), ha, hb, _su(ha)], w2cat[...]) + b2[...][None, None, :]

    h = _resblock(h, er1c[...], er1w2[...])
    h = _resblock(h, er2c[...], er2w2[...])

    z = _mm(h, pvw[...]) + pvb[...][None, None, :]

    zsq = _rowsq(z)
    sc = _mm(z, embT[...])
    dist = (zsq + emb_sq[...][None, :, :]) - 2.0 * sc
    lanes = jax.lax.broadcasted_iota(jnp.int32, (S, L, K), 2)
    dmin = jnp.min(dist, axis=-1, keepdims=True)
    idx = jnp.min(jnp.where(dist == dmin, lanes, K), axis=-1).astype(jnp.int32)
    idx_o[0] = idx

    oh = (lanes == idx[:, :, None]).astype(jnp.float32)
    q = _mm_exact(oh, emb[...])

    cpart = jnp.sum(oh, axis=(0, 1))[None, :]
    epart = jnp.sum((q - z) ** 2)

    @pl.when(i == 0)
    def _init():
        counts_acc[...] = cpart
        err_acc[0, 0] = epart

    @pl.when(i > 0)
    def _acc():
        counts_acc[...] = counts_acc[...] + cpart
        err_acc[0, 0] = err_acc[0, 0] + epart

    d = _imm([_sd(q), q, _su(q)], d1c[...]) + d1b[...][None, None, :]

    d = _resblock(d, dr1c[...], dr1w2[...])
    d = _resblock(d, dr2c[...], dr2w2[...])

    bt1 = t1b[...][None, None, :]
    he = _imm([_sd(d), d], t1e[...]) + bt1
    ho = _imm([d, _su(d)], t1o[...]) + bt1
    he = jax.nn.relu(he)
    ho = jax.nn.relu(ho)

    bias = t2b[0]
    p0_o[0] = _imm([_sd(ho), he], t2e[...])[..., 0] + bias
    p1_o[0] = _imm([he, ho], t2o[...])[..., 0] + bias
    p2_o[0] = _imm([he, ho], t2e[...])[..., 0] + bias
    p3_o[0] = _imm([ho, _su(he)], t2o[...])[..., 0] + bias

    @pl.when(i == GRID - 1)
    def _fin():
        n = jnp.float32(B * L)
        mse = err_acc[0, 0] / (n * D)
        loss_o[0, 0] = 1.25 * mse
        avg = counts_acc[...] / n
        ent = jnp.sum(avg * jnp.log(avg + 1e-10))
        perp_o[0, 0] = jnp.exp(-ent)


@functools.partial(jax.jit, static_argnames=())
def _run(x, params):
    xp = x[:, 0, :].reshape(B, L, 4)
    x0, x1, x2, x3 = (xp[..., k].reshape(GRID, S, L) for k in range(4))

    def taps(w):
        return jnp.transpose(w, (2, 1, 0))

    def cat(w):
        t = taps(w)
        return t.reshape(t.shape[0] * t.shape[1], t.shape[2])

    w1taps = jnp.transpose(params['enc_conv1_w'][:, 0, :], (1, 0))
    w2cat = cat(params['enc_conv2_w'])
    er1c = cat(params['enc_res1_w1'])
    er1w2 = jnp.transpose(params['enc_res1_w2'][:, :, 0], (1, 0))
    er2c = cat(params['enc_res2_w1'])
    er2w2 = jnp.transpose(params['enc_res2_w2'][:, :, 0], (1, 0))
    pvw = jnp.transpose(params['pre_vq_w'][:, :, 0], (1, 0))
    emb = params['emb']
    embT = jnp.transpose(emb, (1, 0))
    esq_sq = emb * emb
    es = esq_sq[:, 0:8]
    for g in range(1, 8):
        es = es + esq_sq[:, 8 * g:8 * (g + 1)]
    et = es[:, 0:4] + es[:, 4:8]
    et = et[:, 0:2] + et[:, 2:4]
    emb_sq = (et[:, 0:1] + et[:, 1:2]).reshape(1, K)

    d1c = cat(params['dec_conv1_w'])
    dr1c = cat(params['dec_res1_w1'])
    dr1w2 = jnp.transpose(params['dec_res1_w2'][:, :, 0], (1, 0))
    dr2c = cat(params['dec_res2_w1'])
    dr2w2 = jnp.transpose(params['dec_res2_w2'][:, :, 0], (1, 0))
    t1 = taps(params['dec_t1_w'])
    t1e = jnp.concatenate([t1[3], t1[1]], axis=0)
    t1o = jnp.concatenate([t1[2], t1[0]], axis=0)
    t2 = taps(params['dec_t2_w'])
    t2e = jnp.concatenate([t2[3], t2[1]], axis=0)
    t2o = jnp.concatenate([t2[2], t2[0]], axis=0)

    def full(a):
        return pl.BlockSpec(a.shape, lambda i: (0,) * a.ndim)

    weights = (w1taps, params['enc_conv1_b'], w2cat, params['enc_conv2_b'],
               er1c, er1w2, er2c, er2w2,
               pvw, params['pre_vq_b'], embT, emb, emb_sq,
               d1c, params['dec_conv1_b'], dr1c, dr1w2, dr2c, dr2w2,
               t1e, t1o, params['dec_t1_b'], t2e, t2o, params['dec_t2_b'])

    xspec = pl.BlockSpec((1, S, L), lambda i: (i, 0, 0))
    in_specs = [xspec] * 4 + [full(w) for w in weights]

    out_shapes = (
        jax.ShapeDtypeStruct((1, 1), jnp.float32),
        jax.ShapeDtypeStruct((1, 1), jnp.float32),
        jax.ShapeDtypeStruct((GRID, S, L), jnp.int32),
        jax.ShapeDtypeStruct((GRID, S, L), jnp.float32),
        jax.ShapeDtypeStruct((GRID, S, L), jnp.float32),
        jax.ShapeDtypeStruct((GRID, S, L), jnp.float32),
        jax.ShapeDtypeStruct((GRID, S, L), jnp.float32),
    )
    out_specs = (
        pl.BlockSpec(memory_space=pltpu.SMEM),
        pl.BlockSpec(memory_space=pltpu.SMEM),
        xspec, xspec, xspec, xspec, xspec,
    )

    loss, perp, idx, p0, p1, p2, p3 = pl.pallas_call(
        _vqvae_kernel,
        grid=(GRID,),
        in_specs=in_specs,
        out_specs=out_specs,
        out_shape=out_shapes,
        scratch_shapes=[
            pltpu.VMEM((1, K), jnp.float32),
            pltpu.SMEM((1, 1), jnp.float32),
        ],
    )(x0, x1, x2, x3, *weights)

    x_recon = jnp.stack(
        [p0.reshape(B, L), p1.reshape(B, L), p2.reshape(B, L),
         p3.reshape(B, L)], axis=-1).reshape(B, 1, T)
    return (loss.reshape(()), x_recon, perp.reshape(()), idx.reshape(B, L))


def kernel(x, params):
    return _run(x, params)


# speedup vs baseline: 1.0521x; 1.0521x over previous
import functools

import jax
import jax.numpy as jnp
from jax.experimental import pallas as pl
from jax.experimental.pallas import tpu as pltpu

B, T, L = 64, 4096, 1024
H, RH, K, D = 128, 64, 1024, 64
S = 2
GRID = B // S


def _mm(a, w):
    return jax.lax.dot_general(
        a.astype(jnp.bfloat16), w.astype(jnp.bfloat16),
        (((a.ndim - 1,), (0,)), ((), ())),
        preferred_element_type=jnp.float32)


def _mm_exact(a, w):
    return jax.lax.dot_general(
        a, w, (((a.ndim - 1,), (0,)), ((), ())),
        preferred_element_type=jnp.float32,
        precision=jax.lax.Precision.HIGHEST)


def _imm(parts, wcat):
    return _mm(jnp.concatenate([p.astype(jnp.bfloat16) for p in parts],
                               axis=-1), wcat)


def _sd(v):
    z = jnp.zeros_like(v[:, :1])
    return jnp.concatenate([z, v[:, :-1]], axis=1)


def _su(v):
    z = jnp.zeros_like(v[:, :1])
    return jnp.concatenate([v[:, 1:], z], axis=1)


def _resblock(x, wcat3, w2):
    r = jax.nn.relu(x).astype(jnp.bfloat16)
    h = _imm([_sd(r), r, _su(r)], wcat3)
    h = jax.nn.relu(h)
    return x + _mm(h, w2)


def _rowsq(z):
    sq = z * z
    s = sq[..., 0:8]
    for g in range(1, 8):
        s = s + sq[..., 8 * g:8 * (g + 1)]
    t = s[..., 0:4] + s[..., 4:8]
    t = t[..., 0:2] + t[..., 2:4]
    return t[..., 0:1] + t[..., 1:2]


def _vqvae_kernel(x0, x1, x2, x3,
                  w1taps, b1, w2cat, b2,
                  er1c, er1w2, er2c, er2w2,
                  pvw, pvb, embT, emb, emb_sq,
                  d1c, d1b, dr1c, dr1w2, dr2c, dr2w2,
                  t1e, t1o, t1b, t2e, t2o, t2b,
                  loss_o, perp_o, idx_o, p0_o, p1_o, p2_o, p3_o,
                  counts_acc, err_acc):
    i = pl.program_id(0)

    xa0, xa1, xa2, xa3 = x0[0], x1[0], x2[0], x3[0]
    w1 = w1taps[...]
    bb1 = b1[...][None, None, :]
    xb0, xb1, xb2, xb3 = (v.astype(jnp.bfloat16) for v in (xa0, xa1, xa2, xa3))
    ha = _mm(jnp.stack([_sd(xb3), xb0, xb1, xb2], axis=-1), w1) + bb1
    hb = _mm(jnp.stack([xb1, xb2, xb3, _su(xb0)], axis=-1), w1) + bb1
    ha = jax.nn.relu(ha)
    hb = jax.nn.relu(hb)

    hab = ha.astype(jnp.bfloat16)
    hbb = hb.astype(jnp.bfloat16)
    h = _imm([_sd(hbb), hab, hbb, _su(hab)], w2cat[...]) + b2[...][None, None, :]

    h = _resblock(h, er1c[...], er1w2[...])
    h = _resblock(h, er2c[...], er2w2[...])

    z = _mm(h, pvw[...]) + pvb[...][None, None, :]

    zsq = _rowsq(z)
    sc = _mm(z, embT[...])
    dist = (zsq + emb_sq[...][None, :, :]) - 2.0 * sc
    lanes = jax.lax.broadcasted_iota(jnp.int32, (S, L, K), 2)
    dmin = jnp.min(dist, axis=-1, keepdims=True)
    idx = jnp.min(jnp.where(dist == dmin, lanes, K), axis=-1).astype(jnp.int32)
    idx_o[0] = idx

    oh = (lanes == idx[:, :, None]).astype(jnp.float32)
    q = _mm_exact(oh, emb[...])

    cpart = jnp.sum(oh, axis=(0, 1))[None, :]
    epart = jnp.sum((q - z) ** 2)

    @pl.when(i == 0)
    def _init():
        counts_acc[...] = cpart
        err_acc[0, 0] = epart

    @pl.when(i > 0)
    def _acc():
        counts_acc[...] = counts_acc[...] + cpart
        err_acc[0, 0] = err_acc[0, 0] + epart

    qb = q.astype(jnp.bfloat16)
    d = _imm([_sd(qb), qb, _su(qb)], d1c[...]) + d1b[...][None, None, :]

    d = _resblock(d, dr1c[...], dr1w2[...])
    d = _resblock(d, dr2c[...], dr2w2[...])

    bt1 = t1b[...][None, None, :]
    db = d.astype(jnp.bfloat16)
    he = _imm([_sd(db), db], t1e[...]) + bt1
    ho = _imm([db, _su(db)], t1o[...]) + bt1
    he = jax.nn.relu(he)
    ho = jax.nn.relu(ho)

    bias = t2b[0]
    heb = he.astype(jnp.bfloat16)
    hob = ho.astype(jnp.bfloat16)
    p0_o[0] = _imm([_sd(hob), heb], t2e[...])[..., 0] + bias
    p1_o[0] = _imm([heb, hob], t2o[...])[..., 0] + bias
    p2_o[0] = _imm([heb, hob], t2e[...])[..., 0] + bias
    p3_o[0] = _imm([hob, _su(heb)], t2o[...])[..., 0] + bias

    @pl.when(i == GRID - 1)
    def _fin():
        n = jnp.float32(B * L)
        mse = err_acc[0, 0] / (n * D)
        loss_o[0, 0] = 1.25 * mse
        avg = counts_acc[...] / n
        ent = jnp.sum(avg * jnp.log(avg + 1e-10))
        perp_o[0, 0] = jnp.exp(-ent)


@functools.partial(jax.jit, static_argnames=())
def _run(x, params):
    xp = x[:, 0, :].reshape(B, L, 4)
    x0, x1, x2, x3 = (xp[..., k].reshape(GRID, S, L) for k in range(4))

    def taps(w):
        return jnp.transpose(w, (2, 1, 0))

    def cat(w):
        t = taps(w)
        return t.reshape(t.shape[0] * t.shape[1], t.shape[2])

    w1taps = jnp.transpose(params['enc_conv1_w'][:, 0, :], (1, 0))
    w2cat = cat(params['enc_conv2_w'])
    er1c = cat(params['enc_res1_w1'])
    er1w2 = jnp.transpose(params['enc_res1_w2'][:, :, 0], (1, 0))
    er2c = cat(params['enc_res2_w1'])
    er2w2 = jnp.transpose(params['enc_res2_w2'][:, :, 0], (1, 0))
    pvw = jnp.transpose(params['pre_vq_w'][:, :, 0], (1, 0))
    emb = params['emb']
    embT = jnp.transpose(emb, (1, 0))
    esq_sq = emb * emb
    es = esq_sq[:, 0:8]
    for g in range(1, 8):
        es = es + esq_sq[:, 8 * g:8 * (g + 1)]
    et = es[:, 0:4] + es[:, 4:8]
    et = et[:, 0:2] + et[:, 2:4]
    emb_sq = (et[:, 0:1] + et[:, 1:2]).reshape(1, K)

    d1c = cat(params['dec_conv1_w'])
    dr1c = cat(params['dec_res1_w1'])
    dr1w2 = jnp.transpose(params['dec_res1_w2'][:, :, 0], (1, 0))
    dr2c = cat(params['dec_res2_w1'])
    dr2w2 = jnp.transpose(params['dec_res2_w2'][:, :, 0], (1, 0))
    t1 = taps(params['dec_t1_w'])
    t1e = jnp.concatenate([t1[3], t1[1]], axis=0)
    t1o = jnp.concatenate([t1[2], t1[0]], axis=0)
    t2 = taps(params['dec_t2_w'])
    t2e = jnp.concatenate([t2[3], t2[1]], axis=0)
    t2o = jnp.concatenate([t2[2], t2[0]], axis=0)

    def full(a):
        return pl.BlockSpec(a.shape, lambda i: (0,) * a.ndim)

    bf = lambda w: w.astype(jnp.bfloat16)
    weights = (bf(w1taps), params['enc_conv1_b'], bf(w2cat),
               params['enc_conv2_b'],
               bf(er1c), bf(er1w2), bf(er2c), bf(er2w2),
               bf(pvw), params['pre_vq_b'], bf(embT), emb, emb_sq,
               bf(d1c), params['dec_conv1_b'], bf(dr1c), bf(dr1w2),
               bf(dr2c), bf(dr2w2),
               bf(t1e), bf(t1o), params['dec_t1_b'], bf(t2e), bf(t2o),
               params['dec_t2_b'])

    xspec = pl.BlockSpec((1, S, L), lambda i: (i, 0, 0))
    in_specs = [xspec] * 4 + [full(w) for w in weights]

    out_shapes = (
        jax.ShapeDtypeStruct((1, 1), jnp.float32),
        jax.ShapeDtypeStruct((1, 1), jnp.float32),
        jax.ShapeDtypeStruct((GRID, S, L), jnp.int32),
        jax.ShapeDtypeStruct((GRID, S, L), jnp.float32),
        jax.ShapeDtypeStruct((GRID, S, L), jnp.float32),
        jax.ShapeDtypeStruct((GRID, S, L), jnp.float32),
        jax.ShapeDtypeStruct((GRID, S, L), jnp.float32),
    )
    out_specs = (
        pl.BlockSpec(memory_space=pltpu.SMEM),
        pl.BlockSpec(memory_space=pltpu.SMEM),
        xspec, xspec, xspec, xspec, xspec,
    )

    loss, perp, idx, p0, p1, p2, p3 = pl.pallas_call(
        _vqvae_kernel,
        grid=(GRID,),
        in_specs=in_specs,
        out_specs=out_specs,
        out_shape=out_shapes,
        scratch_shapes=[
            pltpu.VMEM((1, K), jnp.float32),
            pltpu.SMEM((1, 1), jnp.float32),
        ],
    )(x0, x1, x2, x3, *weights)

    x_recon = jnp.stack(
        [p0.reshape(B, L), p1.reshape(B, L), p2.reshape(B, L),
         p3.reshape(B, L)], axis=-1).reshape(B, 1, T)
    return (loss.reshape(()), x_recon, perp.reshape(()), idx.reshape(B, L))


def kernel(x, params):
    return _run(x, params)
